# baseline (speedup 1.0000x reference)
import jax
import jax.numpy as jnp
from jax.experimental import pallas as pl
from jax.experimental.pallas import tpu as pltpu

_BM = 80
_CACHE_BLOCKS = 31


def _s1_kernel(x_ref, w1_ref, o_ref):
    o_ref[...] = jnp.dot(x_ref[...], w1_ref[...],
                         preferred_element_type=jnp.float32
                         ).astype(jnp.bfloat16)


def _fused_kernel(adj_ref, s1_ref, b1_ref, w2_ref, b2_ref,
                  o_ref, t2_ref, cache_ref):
    p = pl.program_id(0)
    i = pl.program_id(1)
    bm = o_ref.shape[0]
    c2 = 2 * _CACHE_BLOCKS
    is_cached = (i % 2 == 1) & (i < c2)

    @pl.when(p == 0)
    def _layer1():
        a = adj_ref[...].astype(jnp.bfloat16)
        acc = jnp.dot(a, s1_ref[...], preferred_element_type=jnp.float32)
        h = jnp.maximum(acc + b1_ref[...], 0.0)
        t2_ref[pl.ds(i * bm, bm), :] = jnp.dot(
            h, w2_ref[...], preferred_element_type=jnp.float32
        ).astype(jnp.bfloat16)

        @pl.when(is_cached)
        def _stash():
            cache_ref[i // 2] = a

    def _layer2_epilogue(a):
        z = jnp.dot(a, t2_ref[...],
                    preferred_element_type=jnp.float32) + b2_ref[...]
        m = jnp.max(z, axis=1, keepdims=True)
        e = jnp.exp(z - m)
        lse = jnp.log(jnp.sum(e, axis=1, keepdims=True)) + m
        o_ref[...] = z - lse

    @pl.when((p == 1) & is_cached)
    def _layer2_cached():
        _layer2_epilogue(cache_ref[i // 2])

    @pl.when((p == 1) & jnp.logical_not(is_cached))
    def _layer2_streamed():
        _layer2_epilogue(adj_ref[...].astype(jnp.bfloat16))


def kernel(x, adj, W1, b1, W2, b2):
    n, nfeat = x.shape
    nhid = W1.shape[1]
    ncls = W2.shape[1]
    bm = _BM
    c2 = 2 * _CACHE_BLOCKS
    b1r = b1.reshape(1, nhid)
    b2r = b2.reshape(1, ncls)
    BS1 = 2000

    s1 = pl.pallas_call(
        _s1_kernel,
        grid=(n // BS1,),
        in_specs=[pl.BlockSpec((BS1, nfeat), lambda i: (i, 0)),
                  pl.BlockSpec((nfeat, nhid), lambda i: (0, 0))],
        out_specs=pl.BlockSpec((BS1, nhid), lambda i: (i, 0)),
        out_shape=jax.ShapeDtypeStruct((n, nhid), jnp.bfloat16),
    )(x, W1)

    def adj_map(p, i):
        parked = jnp.where((p == 1) & (i % 2 == 1) & (i < c2), i - 1, i)
        return (parked, 0)

    return pl.pallas_call(
        _fused_kernel,
        grid=(2, n // bm),
        in_specs=[pl.BlockSpec((bm, n), adj_map),
                  pl.BlockSpec((n, nhid), lambda p, i: (0, 0)),
                  pl.BlockSpec((1, nhid), lambda p, i: (0, 0)),
                  pl.BlockSpec((nhid, ncls), lambda p, i: (0, 0)),
                  pl.BlockSpec((1, ncls), lambda p, i: (0, 0))],
        out_specs=pl.BlockSpec((bm, ncls), lambda p, i: (i, 0)),
        out_shape=jax.ShapeDtypeStruct((n, ncls), jnp.float32),
        scratch_shapes=[pltpu.VMEM((n, ncls), jnp.bfloat16),
                        pltpu.VMEM((_CACHE_BLOCKS, bm, n), jnp.bfloat16)],
        compiler_params=pltpu.CompilerParams(
            dimension_semantics=("arbitrary", "arbitrary"),
            vmem_limit_bytes=64 * 1024 * 1024),
    )(adj, s1, b1r, W2, b2r)


# speedup vs baseline: 1.0025x; 1.0025x over previous
import jax
import jax.numpy as jnp
from jax.experimental import pallas as pl
from jax.experimental.pallas import tpu as pltpu

_BM = 80
_CACHE_BLOCKS = 31


def _s1_kernel(x_ref, w1_ref, o_ref):
    o_ref[...] = jnp.dot(x_ref[...], w1_ref[...],
                         preferred_element_type=jnp.float32
                         ).astype(jnp.bfloat16)


def _fused_kernel(adj_ref, s1_ref, b1_ref, w2_ref, b2_ref,
                  o_ref, t2_ref, cache_ref):
    p = pl.program_id(0)
    i = pl.program_id(1)
    bm = o_ref.shape[0]
    c2 = 2 * _CACHE_BLOCKS
    is_cached = (i % 2 == 1) & (i < c2)

    @pl.when(p == 0)
    def _layer1():
        @pl.when(is_cached)
        def _stash():
            cache_ref[i // 2] = adj_ref[...].astype(jnp.bfloat16)

        acc = jnp.dot(adj_ref[...].astype(jnp.bfloat16), s1_ref[...],
                      preferred_element_type=jnp.float32)
        h = jnp.maximum(acc + b1_ref[...], 0.0)
        t2_ref[pl.ds(i * bm, bm), :] = jnp.dot(
            h, w2_ref[...], preferred_element_type=jnp.float32
        ).astype(jnp.bfloat16)

    def _layer2_epilogue(a):
        z = jnp.dot(a, t2_ref[...],
                    preferred_element_type=jnp.float32) + b2_ref[...]
        m = jnp.max(z, axis=1, keepdims=True)
        e = jnp.exp(z - m)
        lse = jnp.log(jnp.sum(e, axis=1, keepdims=True)) + m
        o_ref[...] = z - lse

    @pl.when((p == 1) & is_cached)
    def _layer2_cached():
        _layer2_epilogue(cache_ref[i // 2])

    @pl.when((p == 1) & jnp.logical_not(is_cached))
    def _layer2_streamed():
        _layer2_epilogue(adj_ref[...].astype(jnp.bfloat16))


def kernel(x, adj, W1, b1, W2, b2):
    n, nfeat = x.shape
    nhid = W1.shape[1]
    ncls = W2.shape[1]
    bm = _BM
    c2 = 2 * _CACHE_BLOCKS
    b1r = b1.reshape(1, nhid)
    b2r = b2.reshape(1, ncls)
    BS1 = 2000

    s1 = pl.pallas_call(
        _s1_kernel,
        grid=(n // BS1,),
        in_specs=[pl.BlockSpec((BS1, nfeat), lambda i: (i, 0)),
                  pl.BlockSpec((nfeat, nhid), lambda i: (0, 0))],
        out_specs=pl.BlockSpec((BS1, nhid), lambda i: (i, 0)),
        out_shape=jax.ShapeDtypeStruct((n, nhid), jnp.bfloat16),
    )(x, W1)

    def adj_map(p, i):
        parked = jnp.where((p == 1) & (i % 2 == 1) & (i < c2), i - 1, i)
        return (parked, 0)

    return pl.pallas_call(
        _fused_kernel,
        grid=(2, n // bm),
        in_specs=[pl.BlockSpec((bm, n), adj_map),
                  pl.BlockSpec((n, nhid), lambda p, i: (0, 0)),
                  pl.BlockSpec((1, nhid), lambda p, i: (0, 0)),
                  pl.BlockSpec((nhid, ncls), lambda p, i: (0, 0)),
                  pl.BlockSpec((1, ncls), lambda p, i: (0, 0))],
        out_specs=pl.BlockSpec((bm, ncls), lambda p, i: (i, 0)),
        out_shape=jax.ShapeDtypeStruct((n, ncls), jnp.float32),
        scratch_shapes=[pltpu.VMEM((n, ncls), jnp.bfloat16),
                        pltpu.VMEM((_CACHE_BLOCKS, bm, n), jnp.bfloat16)],
        compiler_params=pltpu.CompilerParams(
            dimension_semantics=("arbitrary", "arbitrary"),
            vmem_limit_bytes=64 * 1024 * 1024),
    )(adj, s1, b1r, W2, b2r)


# speedup vs baseline: 1.4096x; 1.4062x over previous
import jax
import jax.numpy as jnp
from jax.experimental import pallas as pl
from jax.experimental.pallas import tpu as pltpu

_BM = 200
_CACHE_BLOCKS = 9


def _s1_kernel(x_ref, w1_ref, o_ref):
    o_ref[...] = jnp.dot(x_ref[...], w1_ref[...],
                         preferred_element_type=jnp.float32
                         ).astype(jnp.bfloat16)


def _fused_kernel(adj_ref, s1_ref, b1_ref, w2_ref, b2_ref,
                  o_ref, t2_ref, cache_ref):
    p = pl.program_id(0)
    i = pl.program_id(1)
    bm = o_ref.shape[0]
    c2 = 2 * _CACHE_BLOCKS
    is_cached = (i % 2 == 1) & (i < c2)

    @pl.when(p == 0)
    def _layer1():
        @pl.when(is_cached)
        def _stash():
            cache_ref[i // 2] = adj_ref[...].astype(jnp.bfloat16)

        acc = jnp.dot(adj_ref[...].astype(jnp.bfloat16), s1_ref[...],
                      preferred_element_type=jnp.float32)
        h = jnp.maximum(acc + b1_ref[...], 0.0)
        t2_ref[pl.ds(i * bm, bm), :] = jnp.dot(
            h, w2_ref[...], preferred_element_type=jnp.float32)

    def _layer2_epilogue(a):
        z = jnp.dot(a, t2_ref[...].astype(jnp.bfloat16),
                    preferred_element_type=jnp.float32) + b2_ref[...]
        m = jnp.max(z, axis=1, keepdims=True)
        e = jnp.exp(z - m)
        lse = jnp.log(jnp.sum(e, axis=1, keepdims=True)) + m
        o_ref[...] = z - lse

    @pl.when((p == 1) & is_cached)
    def _layer2_cached():
        _layer2_epilogue(cache_ref[i // 2])

    @pl.when((p == 1) & jnp.logical_not(is_cached))
    def _layer2_streamed():
        _layer2_epilogue(adj_ref[...].astype(jnp.bfloat16))


def kernel(x, adj, W1, b1, W2, b2):
    n, nfeat = x.shape
    nhid = W1.shape[1]
    ncls = W2.shape[1]
    bm = _BM
    c2 = 2 * _CACHE_BLOCKS
    b1r = b1.reshape(1, nhid)
    b2r = b2.reshape(1, ncls)
    BS1 = 2000

    s1 = pl.pallas_call(
        _s1_kernel,
        grid=(n // BS1,),
        in_specs=[pl.BlockSpec((BS1, nfeat), lambda i: (i, 0)),
                  pl.BlockSpec((nfeat, nhid), lambda i: (0, 0))],
        out_specs=pl.BlockSpec((BS1, nhid), lambda i: (i, 0)),
        out_shape=jax.ShapeDtypeStruct((n, nhid), jnp.bfloat16),
    )(x, W1)

    def adj_map(p, i):
        parked = jnp.where((p == 1) & (i % 2 == 1) & (i < c2), i - 1, i)
        return (parked, 0)

    return pl.pallas_call(
        _fused_kernel,
        grid=(2, n // bm),
        in_specs=[pl.BlockSpec((bm, n), adj_map),
                  pl.BlockSpec((n, nhid), lambda p, i: (0, 0)),
                  pl.BlockSpec((1, nhid), lambda p, i: (0, 0)),
                  pl.BlockSpec((nhid, ncls), lambda p, i: (0, 0)),
                  pl.BlockSpec((1, ncls), lambda p, i: (0, 0))],
        out_specs=pl.BlockSpec((bm, ncls), lambda p, i: (i, 0)),
        out_shape=jax.ShapeDtypeStruct((n, ncls), jnp.float32),
        scratch_shapes=[pltpu.VMEM((n, ncls), jnp.float32),
                        pltpu.VMEM((_CACHE_BLOCKS, bm, n), jnp.bfloat16)],
        compiler_params=pltpu.CompilerParams(
            dimension_semantics=("arbitrary", "arbitrary"),
            vmem_limit_bytes=64 * 1024 * 1024),
    )(adj, s1, b1r, W2, b2r)
